# baseline (speedup 1.0000x reference)
import functools

import jax
import jax.numpy as jnp
from jax import lax
from jax.experimental import pallas as pl
from jax.experimental.pallas import tpu as pltpu
from jax.experimental.pallas import tpu_sc as plsc

_D = 64
_H = 50
_L = 16
_NC = 2
_NS = 16
_NW = _NC * _NS
_BAGS_PER_CHUNK = 2
_IDX_PER_CHUNK = _BAGS_PER_CHUNK * _H
_NBUF = 3
_TV = 128


@functools.lru_cache(maxsize=None)
def _make_transpose(V):
    n_full = V // _TV
    v_rem = V - n_full * _TV
    base_t, extra = divmod(n_full, _NW)
    n_groups = (base_t + 2) // 2
    mesh = plsc.VectorSubcoreMesh(core_axis_name="c", subcore_axis_name="s")

    @functools.partial(
        pl.kernel,
        mesh=mesh,
        out_type=jax.ShapeDtypeStruct((V * _D,), jnp.float32),
        scratch_types=[
            [pltpu.VMEM((_D, _TV), jnp.float32) for _ in range(2)],
            [pltpu.VMEM((_TV * _D,), jnp.float32) for _ in range(2)],
            [pltpu.SemaphoreType.DMA for _ in range(2)],
            [pltpu.SemaphoreType.DMA for _ in range(2)],
        ],
        compiler_params=pltpu.CompilerParams(needs_layout_passes=False),
    )
    def tr(wt_hbm, tail_hbm, out_hbm, sbufs, tbufs, rsems, wsems):
        wid = lax.axis_index("s") * _NC + lax.axis_index("c")
        lo = wid * base_t + jnp.minimum(wid, extra)
        n_t = base_t + (wid < extra).astype(jnp.int32)

        iota = lax.broadcasted_iota(jnp.int32, (_L,), 0)
        row_idx = [iota + d0 for d0 in range(0, _D, _L)]
        zeros16 = jnp.zeros((_L,), jnp.int32)

        def start_read(t, b):
            pltpu.async_copy(wt_hbm.at[:, pl.ds(t * _TV, _TV)], sbufs[b],
                             rsems[b])

        def wait_read(b):
            pltpu.make_async_copy(wt_hbm.at[:, pl.ds(0, _TV)], sbufs[b],
                                  rsems[b]).wait()

        def start_write(t, b):
            pltpu.async_copy(tbufs[b], out_hbm.at[pl.ds(t * _TV * _D,
                                                        _TV * _D)], wsems[b])

        def wait_write(b):
            pltpu.make_async_copy(tbufs[b], out_hbm.at[pl.ds(0, _TV * _D)],
                                  wsems[b]).wait()

        def transpose_tile(b, n_v):
            sb, tb = sbufs[b], tbufs[b]

            def vbody(vg, carry):
                for j in range(8):
                    v = vg * 8 + j
                    col = zeros16 + v
                    for kk in range(_D // _L):
                        g = plsc.load_gather(sb, [row_idx[kk], col])
                        tb[pl.ds(v * _D + kk * _L, _L)] = g
                return carry

            lax.fori_loop(0, n_v // 8, vbody, 0)

        start_read(lo, 0)
        start_read(lo + 1, 1)

        def group(g, carry):
            for b in range(2):
                i = g * 2 + b
                t = lo + i

                @pl.when(i >= 2)
                def _():
                    wait_write(b)

                wait_read(b)
                transpose_tile(b, _TV)
                start_write(t, b)

                @pl.when(i + 2 < n_t)
                def _():
                    start_read(t + 2, b)
            return carry

        lax.fori_loop(0, base_t // 2, group, 0)

        @pl.when(n_t > base_t)
        def _():
            wait_write(0)
            wait_read(0)
            transpose_tile(0, _TV)
            start_write(lo + base_t, 0)

        wait_write(0)
        wait_write(1)

        if v_rem:
            @pl.when(wid == _NW - 1)
            def _():
                pltpu.sync_copy(tail_hbm,
                                out_hbm.at[pl.ds(n_full * _TV * _D,
                                                 v_rem * _D)])

    return tr


@functools.lru_cache(maxsize=None)
def _make_bag(B, V):
    bags_per_w = B // _NW
    chunks_per_w = bags_per_w // _BAGS_PER_CHUNK
    n_groups = chunks_per_w // _NBUF
    tail = chunks_per_w - n_groups * _NBUF
    mesh = plsc.VectorSubcoreMesh(core_axis_name="c", subcore_axis_name="s")

    @functools.partial(
        pl.kernel,
        mesh=mesh,
        out_type=jax.ShapeDtypeStruct((B, _D), jnp.float32),
        scratch_types=[
            pltpu.VMEM((chunks_per_w, _IDX_PER_CHUNK), jnp.int32),
            [pltpu.VMEM((_IDX_PER_CHUNK, _D), jnp.float32)
             for _ in range(_NBUF)],
            pltpu.VMEM((bags_per_w, _D), jnp.float32),
            [pltpu.SemaphoreType.DMA for _ in range(_NBUF)],
        ],
        compiler_params=pltpu.CompilerParams(use_tc_tiling_on_sc=False),
    )
    def k(idx_hbm, table_hbm, out_hbm, idx_v, bufs, out_v, sems):
        wid = lax.axis_index("s") * _NC + lax.axis_index("c")
        pltpu.sync_copy(idx_hbm.at[pl.ds(wid * chunks_per_w, chunks_per_w)],
                        idx_v)

        inv = jnp.float32(1.0 / _H)

        def start(c, b):
            pltpu.async_copy(table_hbm.at[idx_v.at[c]], bufs[b], sems[b])

        def wait(c, b):
            pltpu.make_async_copy(table_hbm.at[idx_v.at[c]], bufs[b],
                                  sems[b]).wait()

        def reduce_chunk(c, b):
            rows_v = bufs[b]
            for bag in range(_BAGS_PER_CHUNK):
                base = bag * _H
                accs = [rows_v[base, pl.ds(kk * _L, _L)]
                        for kk in range(_D // _L)]
                for r in range(1, _H):
                    for kk in range(_D // _L):
                        accs[kk] = accs[kk] + rows_v[base + r,
                                                     pl.ds(kk * _L, _L)]
                obag = c * _BAGS_PER_CHUNK + bag
                for kk in range(_D // _L):
                    out_v[obag, pl.ds(kk * _L, _L)] = accs[kk] * inv

        for b in range(_NBUF):
            start(b, b)

        def group_body(g, carry):
            for b in range(_NBUF):
                c = g * _NBUF + b
                wait(c, b)
                reduce_chunk(c, b)
                start(c + _NBUF, b)
            return carry

        lax.fori_loop(0, n_groups - 1, group_body, 0)

        for i in range(_NBUF + tail):
            c = (n_groups - 1) * _NBUF + i
            b = i % _NBUF
            wait(c, b)
            reduce_chunk(c, b)
            if i + _NBUF < _NBUF + tail:
                start(c + _NBUF, b)

        pltpu.sync_copy(out_v, out_hbm.at[pl.ds(wid * bags_per_w, bags_per_w)])

    return k


def kernel(input_, weight):
    B, H = input_.shape
    V = weight.shape[0]
    idx2 = input_.reshape(B // _BAGS_PER_CHUNK, _IDX_PER_CHUNK)
    n_full = V // _TV
    tail = weight[n_full * _TV:, :].reshape(-1)
    w_lin = _make_transpose(V)(weight.T, tail).reshape(V, _D)
    return _make_bag(B, V)(idx2, w_lin)


# speedup vs baseline: 2.2186x; 2.2186x over previous
import functools

import jax
import jax.numpy as jnp
from jax import lax
from jax.experimental import pallas as pl
from jax.experimental.pallas import tpu as pltpu
from jax.experimental.pallas import tpu_sc as plsc

_D = 64
_H = 50
_L = 16
_NC = 2
_NS = 16
_NW = _NC * _NS
_BAGS_PER_CHUNK = 2
_IDX_PER_CHUNK = _BAGS_PER_CHUNK * _H
_NBUF = 3
_TV = 128


@functools.lru_cache(maxsize=None)
def _make_transpose(V):
    n_full = V // _TV
    v_rem = V - n_full * _TV
    base_t, extra = divmod(n_full, _NW)
    n_groups = (base_t + 2) // 2
    mesh = plsc.VectorSubcoreMesh(core_axis_name="c", subcore_axis_name="s")

    @functools.partial(
        pl.kernel,
        mesh=mesh,
        out_type=jax.ShapeDtypeStruct((V * _D,), jnp.float32),
        scratch_types=[
            [pltpu.VMEM((_D, _TV), jnp.float32) for _ in range(2)],
            [pltpu.VMEM((_TV * _D,), jnp.float32) for _ in range(2)],
            [pltpu.SemaphoreType.DMA for _ in range(2)],
            [pltpu.SemaphoreType.DMA for _ in range(2)],
        ],
        compiler_params=pltpu.CompilerParams(needs_layout_passes=False),
    )
    def tr(wt_hbm, tail_hbm, out_hbm, sbufs, tbufs, rsems, wsems):
        wid = lax.axis_index("s") * _NC + lax.axis_index("c")
        lo = wid * base_t + jnp.minimum(wid, extra)
        n_t = base_t + (wid < extra).astype(jnp.int32)

        iota = lax.broadcasted_iota(jnp.int32, (_L,), 0)
        row_idx = [iota + d0 for d0 in range(0, _D, _L)]
        qs = [(iota + k) & 15 for k in range(_L)]
        q64s = [qs[k] * _D + iota for k in range(_L)]

        def start_read(t, b):
            pltpu.async_copy(wt_hbm.at[:, pl.ds(t * _TV, _TV)], sbufs[b],
                             rsems[b])

        def wait_read(b):
            pltpu.make_async_copy(wt_hbm.at[:, pl.ds(0, _TV)], sbufs[b],
                                  rsems[b]).wait()

        def start_write(t, b):
            pltpu.async_copy(tbufs[b], out_hbm.at[pl.ds(t * _TV * _D,
                                                        _TV * _D)], wsems[b])

        def wait_write(b):
            pltpu.make_async_copy(tbufs[b], out_hbm.at[pl.ds(0, _TV * _D)],
                                  wsems[b]).wait()

        def transpose_tile(b, n_v):
            sb, tb = sbufs[b], tbufs[b]

            def vbody(vbi, carry):
                vb = vbi * _L
                for d0i in range(_D // _L):
                    d0 = d0i * _L
                    base = vb * _D + d0
                    for k in range(_L):
                        g = plsc.load_gather(sb, [row_idx[d0i], qs[k] + vb])
                        plsc.store_scatter(tb, [q64s[k] + base], g)
                return carry

            lax.fori_loop(0, n_v // _L, vbody, 0)

        start_read(lo, 0)
        start_read(lo + 1, 1)

        def group(g, carry):
            for b in range(2):
                i = g * 2 + b
                t = lo + i

                @pl.when(i >= 2)
                def _():
                    wait_write(b)

                wait_read(b)
                transpose_tile(b, _TV)
                start_write(t, b)

                @pl.when(i + 2 < n_t)
                def _():
                    start_read(t + 2, b)
            return carry

        lax.fori_loop(0, base_t // 2, group, 0)

        @pl.when(n_t > base_t)
        def _():
            wait_write(0)
            wait_read(0)
            transpose_tile(0, _TV)
            start_write(lo + base_t, 0)

        wait_write(0)
        wait_write(1)

        if v_rem:
            @pl.when(wid == _NW - 1)
            def _():
                pltpu.sync_copy(tail_hbm,
                                out_hbm.at[pl.ds(n_full * _TV * _D,
                                                 v_rem * _D)])

    return tr


@functools.lru_cache(maxsize=None)
def _make_bag(B, V):
    bags_per_w = B // _NW
    chunks_per_w = bags_per_w // _BAGS_PER_CHUNK
    n_groups = chunks_per_w // _NBUF
    tail = chunks_per_w - n_groups * _NBUF
    mesh = plsc.VectorSubcoreMesh(core_axis_name="c", subcore_axis_name="s")

    @functools.partial(
        pl.kernel,
        mesh=mesh,
        out_type=jax.ShapeDtypeStruct((B, _D), jnp.float32),
        scratch_types=[
            pltpu.VMEM((chunks_per_w, _IDX_PER_CHUNK), jnp.int32),
            [pltpu.VMEM((_IDX_PER_CHUNK, _D), jnp.float32)
             for _ in range(_NBUF)],
            pltpu.VMEM((bags_per_w, _D), jnp.float32),
            [pltpu.SemaphoreType.DMA for _ in range(_NBUF)],
        ],
        compiler_params=pltpu.CompilerParams(use_tc_tiling_on_sc=False),
    )
    def k(idx_hbm, table_hbm, out_hbm, idx_v, bufs, out_v, sems):
        wid = lax.axis_index("s") * _NC + lax.axis_index("c")
        pltpu.sync_copy(idx_hbm.at[pl.ds(wid * chunks_per_w, chunks_per_w)],
                        idx_v)

        inv = jnp.float32(1.0 / _H)

        def start(c, b):
            pltpu.async_copy(table_hbm.at[idx_v.at[c]], bufs[b], sems[b])

        def wait(c, b):
            pltpu.make_async_copy(table_hbm.at[idx_v.at[c]], bufs[b],
                                  sems[b]).wait()

        def reduce_chunk(c, b):
            rows_v = bufs[b]
            for bag in range(_BAGS_PER_CHUNK):
                base = bag * _H
                accs = [rows_v[base, pl.ds(kk * _L, _L)]
                        for kk in range(_D // _L)]
                for r in range(1, _H):
                    for kk in range(_D // _L):
                        accs[kk] = accs[kk] + rows_v[base + r,
                                                     pl.ds(kk * _L, _L)]
                obag = c * _BAGS_PER_CHUNK + bag
                for kk in range(_D // _L):
                    out_v[obag, pl.ds(kk * _L, _L)] = accs[kk] * inv

        for b in range(_NBUF):
            start(b, b)

        def group_body(g, carry):
            for b in range(_NBUF):
                c = g * _NBUF + b
                wait(c, b)
                reduce_chunk(c, b)
                start(c + _NBUF, b)
            return carry

        lax.fori_loop(0, n_groups - 1, group_body, 0)

        for i in range(_NBUF + tail):
            c = (n_groups - 1) * _NBUF + i
            b = i % _NBUF
            wait(c, b)
            reduce_chunk(c, b)
            if i + _NBUF < _NBUF + tail:
                start(c + _NBUF, b)

        pltpu.sync_copy(out_v, out_hbm.at[pl.ds(wid * bags_per_w, bags_per_w)])

    return k


def kernel(input_, weight):
    B, H = input_.shape
    V = weight.shape[0]
    idx2 = input_.reshape(B // _BAGS_PER_CHUNK, _IDX_PER_CHUNK)
    n_full = V // _TV
    tail = weight[n_full * _TV:, :].reshape(-1)
    w_lin = _make_transpose(V)(weight.T, tail).reshape(V, _D)
    return _make_bag(B, V)(idx2, w_lin)


# speedup vs baseline: 2.9698x; 1.3386x over previous
import functools

import jax
import jax.numpy as jnp
from jax import lax
from jax.experimental import pallas as pl
from jax.experimental.pallas import tpu as pltpu
from jax.experimental.pallas import tpu_sc as plsc

_D = 64
_H = 50
_L = 16
_NC = 2
_NS = 16
_NW = _NC * _NS
_BAGS_PER_CHUNK = 2
_IDX_PER_CHUNK = _BAGS_PER_CHUNK * _H
_NBUF = 3
_TV = 128


@functools.lru_cache(maxsize=None)
def _make_transpose(V):
    n_full = V // _TV
    v_rem = V - n_full * _TV
    base_t, extra = divmod(n_full, _NW)
    n_groups = (base_t + 2) // 2
    mesh = plsc.VectorSubcoreMesh(core_axis_name="c", subcore_axis_name="s")

    @functools.partial(
        pl.kernel,
        mesh=mesh,
        out_type=jax.ShapeDtypeStruct((V * _D,), jnp.float32),
        scratch_types=[
            [pltpu.VMEM((_D, _TV), jnp.float32) for _ in range(2)],
            [pltpu.VMEM((_TV * _D,), jnp.float32) for _ in range(2)],
            [pltpu.SemaphoreType.DMA for _ in range(2)],
            [pltpu.SemaphoreType.DMA for _ in range(2)],
        ],
        compiler_params=pltpu.CompilerParams(needs_layout_passes=False),
    )
    def tr(wt_hbm, tail_hbm, out_hbm, sbufs, tbufs, rsems, wsems):
        wid = lax.axis_index("s") * _NC + lax.axis_index("c")
        lo = wid * base_t + jnp.minimum(wid, extra)
        n_t = base_t + (wid < extra).astype(jnp.int32)

        iota = lax.broadcasted_iota(jnp.int32, (_L,), 0)
        row_idx = [iota + d0 for d0 in range(0, _D, _L)]
        qs = [(iota + k) & 15 for k in range(_L)]
        q64s = [qs[k] * _D + iota for k in range(_L)]

        def start_read(t, b):
            pltpu.async_copy(wt_hbm.at[:, pl.ds(t * _TV, _TV)], sbufs[b],
                             rsems[b])

        def wait_read(b):
            pltpu.make_async_copy(wt_hbm.at[:, pl.ds(0, _TV)], sbufs[b],
                                  rsems[b]).wait()

        def start_write(t, b):
            pltpu.async_copy(tbufs[b], out_hbm.at[pl.ds(t * _TV * _D,
                                                        _TV * _D)], wsems[b])

        def wait_write(b):
            pltpu.make_async_copy(tbufs[b], out_hbm.at[pl.ds(0, _TV * _D)],
                                  wsems[b]).wait()

        def transpose_tile(b, n_v):
            sb, tb = sbufs[b], tbufs[b]

            @plsc.parallel_loop(0, n_v // _L, unroll=2)
            def vbody(vbi):
                vb = vbi * _L
                for k in range(_L):
                    col = qs[k] + vb
                    for d0i in range(_D // _L):
                        g = plsc.load_gather(sb, [row_idx[d0i], col])
                        plsc.store_scatter(
                            tb, [q64s[k] + (vb * _D + d0i * _L)], g)

        start_read(lo, 0)
        start_read(lo + 1, 1)

        def group(g, carry):
            for b in range(2):
                i = g * 2 + b
                t = lo + i

                @pl.when(i >= 2)
                def _():
                    wait_write(b)

                wait_read(b)
                transpose_tile(b, _TV)
                start_write(t, b)

                @pl.when(i + 2 < n_t)
                def _():
                    start_read(t + 2, b)
            return carry

        lax.fori_loop(0, base_t // 2, group, 0)

        @pl.when(n_t > base_t)
        def _():
            wait_write(0)
            wait_read(0)
            transpose_tile(0, _TV)
            start_write(lo + base_t, 0)

        wait_write(0)
        wait_write(1)

        if v_rem:
            @pl.when(wid == _NW - 1)
            def _():
                pltpu.sync_copy(tail_hbm,
                                out_hbm.at[pl.ds(n_full * _TV * _D,
                                                 v_rem * _D)])

    return tr


@functools.lru_cache(maxsize=None)
def _make_bag(B, V):
    bags_per_w = B // _NW
    chunks_per_w = bags_per_w // _BAGS_PER_CHUNK
    n_groups = chunks_per_w // _NBUF
    tail = chunks_per_w - n_groups * _NBUF
    mesh = plsc.VectorSubcoreMesh(core_axis_name="c", subcore_axis_name="s")

    @functools.partial(
        pl.kernel,
        mesh=mesh,
        out_type=jax.ShapeDtypeStruct((B, _D), jnp.float32),
        scratch_types=[
            pltpu.VMEM((chunks_per_w, _IDX_PER_CHUNK), jnp.int32),
            [pltpu.VMEM((_IDX_PER_CHUNK, _D), jnp.float32)
             for _ in range(_NBUF)],
            pltpu.VMEM((bags_per_w, _D), jnp.float32),
            [pltpu.SemaphoreType.DMA for _ in range(_NBUF)],
        ],
        compiler_params=pltpu.CompilerParams(use_tc_tiling_on_sc=False),
    )
    def k(idx_hbm, table_hbm, out_hbm, idx_v, bufs, out_v, sems):
        wid = lax.axis_index("s") * _NC + lax.axis_index("c")
        pltpu.sync_copy(idx_hbm.at[pl.ds(wid * chunks_per_w, chunks_per_w)],
                        idx_v)

        inv = jnp.float32(1.0 / _H)

        def start(c, b):
            pltpu.async_copy(table_hbm.at[idx_v.at[c]], bufs[b], sems[b])

        def wait(c, b):
            pltpu.make_async_copy(table_hbm.at[idx_v.at[c]], bufs[b],
                                  sems[b]).wait()

        def reduce_chunk(c, b):
            rows_v = bufs[b]
            for bag in range(_BAGS_PER_CHUNK):
                base = bag * _H
                accs = [rows_v[base, pl.ds(kk * _L, _L)]
                        for kk in range(_D // _L)]
                for r in range(1, _H):
                    for kk in range(_D // _L):
                        accs[kk] = accs[kk] + rows_v[base + r,
                                                     pl.ds(kk * _L, _L)]
                obag = c * _BAGS_PER_CHUNK + bag
                for kk in range(_D // _L):
                    out_v[obag, pl.ds(kk * _L, _L)] = accs[kk] * inv

        for b in range(_NBUF):
            start(b, b)

        def group_body(g, carry):
            for b in range(_NBUF):
                c = g * _NBUF + b
                wait(c, b)
                reduce_chunk(c, b)
                start(c + _NBUF, b)
            return carry

        lax.fori_loop(0, n_groups - 1, group_body, 0)

        for i in range(_NBUF + tail):
            c = (n_groups - 1) * _NBUF + i
            b = i % _NBUF
            wait(c, b)
            reduce_chunk(c, b)
            if i + _NBUF < _NBUF + tail:
                start(c + _NBUF, b)

        pltpu.sync_copy(out_v, out_hbm.at[pl.ds(wid * bags_per_w, bags_per_w)])

    return k


def kernel(input_, weight):
    B, H = input_.shape
    V = weight.shape[0]
    idx2 = input_.reshape(B // _BAGS_PER_CHUNK, _IDX_PER_CHUNK)
    n_full = V // _TV
    tail = weight[n_full * _TV:, :].reshape(-1)
    w_lin = _make_transpose(V)(weight.T, tail).reshape(V, _D)
    return _make_bag(B, V)(idx2, w_lin)


# speedup vs baseline: 3.4411x; 1.1587x over previous
import functools

import jax
import jax.numpy as jnp
from jax import lax
from jax.experimental import pallas as pl
from jax.experimental.pallas import tpu as pltpu
from jax.experimental.pallas import tpu_sc as plsc

_D = 64
_H = 50
_L = 16
_NC = 2
_NS = 16
_NW = _NC * _NS
_BAGS_PER_CHUNK = 2
_IDX_PER_CHUNK = _BAGS_PER_CHUNK * _H
_NBUF = 3
_TV = 256
_TNB = 3


@functools.lru_cache(maxsize=None)
def _make_transpose(V):
    n_full = V // _TV
    v_rem = V - n_full * _TV
    base_t, extra = divmod(n_full, _NW)
    n_groups = base_t // _TNB
    leftover = base_t - n_groups * _TNB
    mesh = plsc.VectorSubcoreMesh(core_axis_name="c", subcore_axis_name="s")

    @functools.partial(
        pl.kernel,
        mesh=mesh,
        out_type=jax.ShapeDtypeStruct((V * _D,), jnp.float32),
        scratch_types=[
            [pltpu.VMEM((_D, _TV), jnp.float32) for _ in range(_TNB)],
            [pltpu.VMEM((_TV * _D,), jnp.float32) for _ in range(_TNB)],
            [pltpu.SemaphoreType.DMA for _ in range(_TNB)],
            [pltpu.SemaphoreType.DMA for _ in range(_TNB)],
        ],
        compiler_params=pltpu.CompilerParams(needs_layout_passes=False),
    )
    def tr(wt_hbm, tail_hbm, out_hbm, sbufs, tbufs, rsems, wsems):
        wid = lax.axis_index("s") * _NC + lax.axis_index("c")
        lo = wid * base_t + jnp.minimum(wid, extra)
        n_t = base_t + (wid < extra).astype(jnp.int32)

        iota = lax.broadcasted_iota(jnp.int32, (_L,), 0)
        row_idx = [iota + d0 for d0 in range(0, _D, _L)]
        qs = [(iota + k) & 15 for k in range(_L)]
        q64s = [qs[k] * _D + iota for k in range(_L)]

        def start_read(t, b):
            pltpu.async_copy(wt_hbm.at[:, pl.ds(t * _TV, _TV)], sbufs[b],
                             rsems[b])

        def wait_read(b):
            pltpu.make_async_copy(wt_hbm.at[:, pl.ds(0, _TV)], sbufs[b],
                                  rsems[b]).wait()

        def start_write(t, b):
            pltpu.async_copy(tbufs[b], out_hbm.at[pl.ds(t * _TV * _D,
                                                        _TV * _D)], wsems[b])

        def wait_write(b):
            pltpu.make_async_copy(tbufs[b], out_hbm.at[pl.ds(0, _TV * _D)],
                                  wsems[b]).wait()

        def transpose_tile(b, n_v):
            sb, tb = sbufs[b], tbufs[b]

            @plsc.parallel_loop(0, n_v // _L)
            def vbody(vbi):
                vb = vbi * _L
                for k in range(_L):
                    col = qs[k] + vb
                    for d0i in range(_D // _L):
                        g = plsc.load_gather(sb, [row_idx[d0i], col])
                        plsc.store_scatter(
                            tb, [q64s[k] + (vb * _D + d0i * _L)], g)

        for b in range(_TNB):
            start_read(lo + b, b)

        def group(g, carry):
            for b in range(_TNB):
                i = g * _TNB + b
                t = lo + i

                @pl.when(i >= _TNB)
                def _():
                    wait_write(b)

                wait_read(b)
                transpose_tile(b, _TV)
                start_write(t, b)

                @pl.when(i + _TNB < n_t)
                def _():
                    start_read(t + _TNB, b)
            return carry

        lax.fori_loop(0, n_groups, group, 0)

        for j in range(leftover):
            i = n_groups * _TNB + j
            b = i % _TNB
            wait_write(b)
            wait_read(b)
            transpose_tile(b, _TV)
            start_write(lo + i, b)

        b_x = base_t % _TNB

        @pl.when(n_t > base_t)
        def _():
            wait_write(b_x)
            wait_read(b_x)
            transpose_tile(b_x, _TV)
            start_write(lo + base_t, b_x)

        for b in range(_TNB):
            wait_write(b)

        if v_rem:
            @pl.when(wid == _NW - 1)
            def _():
                pltpu.sync_copy(tail_hbm,
                                out_hbm.at[pl.ds(n_full * _TV * _D,
                                                 v_rem * _D)])

    return tr


@functools.lru_cache(maxsize=None)
def _make_bag(B, V):
    bags_per_w = B // _NW
    chunks_per_w = bags_per_w // _BAGS_PER_CHUNK
    n_groups = chunks_per_w // _NBUF
    tail = chunks_per_w - n_groups * _NBUF
    mesh = plsc.VectorSubcoreMesh(core_axis_name="c", subcore_axis_name="s")

    @functools.partial(
        pl.kernel,
        mesh=mesh,
        out_type=jax.ShapeDtypeStruct((B, _D), jnp.float32),
        scratch_types=[
            pltpu.VMEM((chunks_per_w, _IDX_PER_CHUNK), jnp.int32),
            [pltpu.VMEM((_IDX_PER_CHUNK, _D), jnp.float32)
             for _ in range(_NBUF)],
            pltpu.VMEM((bags_per_w, _D), jnp.float32),
            [pltpu.SemaphoreType.DMA for _ in range(_NBUF)],
        ],
        compiler_params=pltpu.CompilerParams(use_tc_tiling_on_sc=False),
    )
    def k(idx_hbm, table_hbm, out_hbm, idx_v, bufs, out_v, sems):
        wid = lax.axis_index("s") * _NC + lax.axis_index("c")
        pltpu.sync_copy(idx_hbm.at[pl.ds(wid * chunks_per_w, chunks_per_w)],
                        idx_v)

        inv = jnp.float32(1.0 / _H)

        def start(c, b):
            pltpu.async_copy(table_hbm.at[idx_v.at[c]], bufs[b], sems[b])

        def wait(c, b):
            pltpu.make_async_copy(table_hbm.at[idx_v.at[c]], bufs[b],
                                  sems[b]).wait()

        def reduce_chunk(c, b):
            rows_v = bufs[b]
            for bag in range(_BAGS_PER_CHUNK):
                base = bag * _H
                accs = [rows_v[base, pl.ds(kk * _L, _L)]
                        for kk in range(_D // _L)]
                for r in range(1, _H):
                    for kk in range(_D // _L):
                        accs[kk] = accs[kk] + rows_v[base + r,
                                                     pl.ds(kk * _L, _L)]
                obag = c * _BAGS_PER_CHUNK + bag
                for kk in range(_D // _L):
                    out_v[obag, pl.ds(kk * _L, _L)] = accs[kk] * inv

        for b in range(_NBUF):
            start(b, b)

        def group_body(g, carry):
            for b in range(_NBUF):
                c = g * _NBUF + b
                wait(c, b)
                reduce_chunk(c, b)
                start(c + _NBUF, b)
            return carry

        lax.fori_loop(0, n_groups - 1, group_body, 0)

        for i in range(_NBUF + tail):
            c = (n_groups - 1) * _NBUF + i
            b = i % _NBUF
            wait(c, b)
            reduce_chunk(c, b)
            if i + _NBUF < _NBUF + tail:
                start(c + _NBUF, b)

        pltpu.sync_copy(out_v, out_hbm.at[pl.ds(wid * bags_per_w, bags_per_w)])

    return k


def kernel(input_, weight):
    B, H = input_.shape
    V = weight.shape[0]
    idx2 = input_.reshape(B // _BAGS_PER_CHUNK, _IDX_PER_CHUNK)
    n_full = V // _TV
    tail = weight[n_full * _TV:, :].reshape(-1)
    w_lin = _make_transpose(V)(weight.T, tail).reshape(V, _D)
    return _make_bag(B, V)(idx2, w_lin)
